# baseline (speedup 1.0000x reference)
import functools
import math

import jax
import jax.numpy as jnp
from jax import lax
from jax.experimental import pallas as pl
from jax.experimental.pallas import tpu as pltpu
from jax.experimental.pallas import tpu_sc as plsc

D = 128
NC, NS = 2, 16
CHUNK = 128
NBUF = 2
IDXBLK = 16
NPART = 1


def _node_tables_kernel(x_ref, w2_ref, w3_ref, t2_ref, t3_ref):
    xb = x_ref[...]
    t2_ref[...] = jnp.dot(xb, w2_ref[...], preferred_element_type=jnp.float32)
    t3_ref[...] = jnp.dot(xb, w3_ref[...], preferred_element_type=jnp.float32)


def _edge_out_kernel(ea_ref, g2_ref, g3_ref, w1_ref, b_ref, o_ref):
    o_ref[...] = (
        jnp.dot(ea_ref[...], w1_ref[...], preferred_element_type=jnp.float32)
        + g2_ref[...] + g3_ref[...] + b_ref[...]
    )


def _edge_out_kernel_acc(prev_ref, ea_ref, g2_ref, g3_ref, w1_ref, b_ref, o_ref):
    del prev_ref
    _edge_out_kernel(ea_ref, g2_ref, g3_ref, w1_ref, b_ref, o_ref)


def _sc_gather(t2, t3, idx2d):
    n_chunks, chunk = idx2d.shape
    assert chunk == CHUNK
    n_idx = n_chunks * chunk
    chunks_per_sub = n_chunks // (NC * NS)
    assert chunks_per_sub % IDXBLK == 0 and IDXBLK % NBUF == 0

    n_nodes = t2.shape[0]
    rows_per_sub = (n_nodes // NS) & ~7
    tail_rows = n_nodes - NS * rows_per_sub
    mesh = plsc.VectorSubcoreMesh(core_axis_name="c", subcore_axis_name="s")

    @functools.partial(
        pl.kernel,
        out_type=jax.ShapeDtypeStruct((n_idx, D), jnp.float32),
        mesh=mesh,
        scratch_types=[
            pltpu.VMEM((2 * IDXBLK, CHUNK), jnp.int32),
            pltpu.VMEM((CHUNK, D), jnp.float32),
            pltpu.VMEM((CHUNK, D), jnp.float32),
            pltpu.VMEM_SHARED((n_nodes, D), jnp.float32),
            pltpu.SemaphoreType.DMA,
            pltpu.SemaphoreType.DMA,
            pltpu.SemaphoreType.DMA,
            pltpu.SemaphoreType.DMA,
            pltpu.SemaphoreType.DMA,
        ],
    )
    def gather_k(t2_hbm, t3_hbm, idx_hbm, g_hbm,
                 idx_v, a0, a1, t_sh, sg0, sg1, sw0, sw1, si):
        cid = lax.axis_index("c")
        sid = lax.axis_index("s")

        roff = sid * rows_per_sub

        @pl.when(cid == 0)
        def _stage_t2():
            pltpu.sync_copy(t2_hbm.at[pl.ds(roff, rows_per_sub)],
                            t_sh.at[pl.ds(roff, rows_per_sub)])

        @pl.when(cid != 0)
        def _stage_t3():
            pltpu.sync_copy(t3_hbm.at[pl.ds(roff, rows_per_sub)],
                            t_sh.at[pl.ds(roff, rows_per_sub)])

        if tail_rows:
            toff = NS * rows_per_sub

            @pl.when((sid == 0) & (cid == 0))
            def _tail_t2():
                pltpu.sync_copy(t2_hbm.at[pl.ds(toff, tail_rows)],
                                t_sh.at[pl.ds(toff, tail_rows)])

            @pl.when((sid == 0) & (cid != 0))
            def _tail_t3():
                pltpu.sync_copy(t3_hbm.at[pl.ds(toff, tail_rows)],
                                t_sh.at[pl.ds(toff, tail_rows)])

        chunk0 = (cid * NS + sid) * chunks_per_sub
        n_blks = chunks_per_sub // IDXBLK
        pltpu.async_copy(idx_hbm.at[pl.ds(chunk0, IDXBLK)],
                         idx_v.at[pl.ds(0, IDXBLK)], si)
        plsc.subcore_barrier()

        slots = ((a0, sg0, sw0), (a1, sg1, sw1))
        steps_per_blk = IDXBLK // NBUF

        def blk_body(blk, carry):
            islot = (blk % 2) * IDXBLK
            pltpu.make_async_copy(idx_hbm.at[pl.ds(0, IDXBLK)],
                                  idx_v.at[pl.ds(0, IDXBLK)], si).wait()

            @pl.when(blk + 1 < n_blks)
            def _prefetch_next():
                nslot = ((blk + 1) % 2) * IDXBLK
                pltpu.async_copy(
                    idx_hbm.at[pl.ds(chunk0 + (blk + 1) * IDXBLK, IDXBLK)],
                    idx_v.at[pl.ds(nslot, IDXBLK)], si)

            def step(p, c2):
                for b, (buf, sg, sw) in enumerate(slots):
                    j = p * NBUF + b

                    @pl.when((blk > 0) | (p > 0))
                    def _wait_prev_write(buf=buf, sw=sw):
                        pltpu.make_async_copy(
                            g_hbm.at[pl.ds(0, CHUNK)], buf, sw).wait()

                    pltpu.async_copy(t_sh.at[idx_v.at[islot + j]], buf, sg)
                for b, (buf, sg, sw) in enumerate(slots):
                    j = p * NBUF + b
                    pltpu.make_async_copy(
                        g_hbm.at[pl.ds(0, CHUNK)], buf, sg).wait()
                    off = (chunk0 + blk * IDXBLK + j) * CHUNK
                    pltpu.async_copy(buf, g_hbm.at[pl.ds(off, CHUNK)], sw)
                return c2

            lax.fori_loop(0, steps_per_blk, step, 0)
            return carry

        lax.fori_loop(0, n_blks, blk_body, 0)
        for buf, _sg, sw in slots:
            pltpu.make_async_copy(g_hbm.at[pl.ds(0, CHUNK)], buf, sw).wait()

    return gather_k(t2, t3, idx2d)


def kernel(x, edge_attr, edge_index, W, b):
    n_nodes, d = x.shape
    n_edges = edge_attr.shape[0]
    senders = edge_index[0].astype(jnp.int32)
    receivers = edge_index[1].astype(jnp.int32)
    W1, W2, W3 = W[:d], W[d:2 * d], W[2 * d:]

    nb = 5
    node_rows = n_nodes // nb
    t2, t3 = pl.pallas_call(
        _node_tables_kernel,
        grid=(nb,),
        in_specs=[
            pl.BlockSpec((node_rows, d), lambda i: (i, 0)),
            pl.BlockSpec((d, d), lambda i: (0, 0)),
            pl.BlockSpec((d, d), lambda i: (0, 0)),
        ],
        out_specs=[
            pl.BlockSpec((node_rows, d), lambda i: (i, 0)),
            pl.BlockSpec((node_rows, d), lambda i: (i, 0)),
        ],
        out_shape=[jax.ShapeDtypeStruct((n_nodes, d), jnp.float32)] * 2,
    )(x, W2, W3)

    eb = 2560 // NPART
    ne_q = n_edges // NPART
    assert ne_q % eb == 0
    lcm = math.lcm(NS * CHUNK * IDXBLK, eb)
    half = -(-ne_q // lcm) * lcm
    pad = half - ne_q
    zpad = jnp.zeros((pad,), jnp.int32)
    nblk_q = ne_q // eb
    hblk = half // eb

    gs = []
    for q in range(NPART):
        s_q = lax.dynamic_slice_in_dim(senders, q * ne_q, ne_q)
        r_q = lax.dynamic_slice_in_dim(receivers, q * ne_q, ne_q)
        idx2d = jnp.concatenate([s_q, zpad, r_q, zpad]).reshape(-1, CHUNK)
        gs.append(_sc_gather(t2, t3, idx2d))

    out = None
    for q in range(NPART):
        g = gs[q]
        qb = q * nblk_q
        in_specs = [
            pl.BlockSpec((eb, d), lambda i, qb=qb: (i + qb, 0)),
            pl.BlockSpec((eb, d), lambda i: (i, 0)),
            pl.BlockSpec((eb, d), lambda i: (i + hblk, 0)),
            pl.BlockSpec((d, d), lambda i: (0, 0)),
            pl.BlockSpec((1, d), lambda i: (0, 0)),
        ]
        out_spec = pl.BlockSpec((eb, d), lambda i, qb=qb: (i + qb, 0))
        out_shape = jax.ShapeDtypeStruct((n_edges, d), jnp.float32)
        args = (edge_attr, g, g, W1, b.reshape(1, d))
        if q == 0:
            out = pl.pallas_call(
                _edge_out_kernel,
                grid=(nblk_q,),
                in_specs=in_specs,
                out_specs=out_spec,
                out_shape=out_shape,
            )(*args)
        else:
            out = pl.pallas_call(
                _edge_out_kernel_acc,
                grid=(nblk_q,),
                in_specs=[pl.BlockSpec(memory_space=pltpu.MemorySpace.HBM)]
                + in_specs,
                out_specs=out_spec,
                out_shape=out_shape,
                input_output_aliases={0: 0},
            )(out, *args)
    return out


# speedup vs baseline: 1.1858x; 1.1858x over previous
import functools
import math

import jax
import jax.numpy as jnp
from jax import lax
from jax.experimental import pallas as pl
from jax.experimental.pallas import tpu as pltpu
from jax.experimental.pallas import tpu_sc as plsc

D = 128
NC, NS = 2, 16
CHUNK = 64
NBUF = 4
IDXBLK = 16
NPART = 1


def _node_tables_kernel(x_ref, w2_ref, w3_ref, t2_ref, t3_ref):
    xb = x_ref[...]
    t2_ref[...] = jnp.dot(xb, w2_ref[...], preferred_element_type=jnp.float32)
    t3_ref[...] = jnp.dot(xb, w3_ref[...], preferred_element_type=jnp.float32)


def _edge_out_kernel(ea_ref, g2_ref, g3_ref, w1_ref, b_ref, o_ref):
    o_ref[...] = (
        jnp.dot(ea_ref[...], w1_ref[...], preferred_element_type=jnp.float32)
        + g2_ref[...] + g3_ref[...] + b_ref[...]
    )


def _edge_out_kernel_acc(prev_ref, ea_ref, g2_ref, g3_ref, w1_ref, b_ref, o_ref):
    del prev_ref
    _edge_out_kernel(ea_ref, g2_ref, g3_ref, w1_ref, b_ref, o_ref)


def _sc_gather(t2, t3, idx2d):
    n_chunks, chunk = idx2d.shape
    assert chunk == CHUNK
    n_idx = n_chunks * chunk
    chunks_per_sub = n_chunks // (NC * NS)
    assert chunks_per_sub % IDXBLK == 0 and IDXBLK % NBUF == 0

    n_nodes = t2.shape[0]
    rows_per_sub = (n_nodes // NS) & ~7
    tail_rows = n_nodes - NS * rows_per_sub
    mesh = plsc.VectorSubcoreMesh(core_axis_name="c", subcore_axis_name="s")

    @functools.partial(
        pl.kernel,
        out_type=jax.ShapeDtypeStruct((n_idx, D), jnp.float32),
        mesh=mesh,
        scratch_types=[
            pltpu.VMEM((2 * IDXBLK, CHUNK), jnp.int32),
            pltpu.VMEM((CHUNK, D), jnp.float32),
            pltpu.VMEM((CHUNK, D), jnp.float32),
            pltpu.VMEM((CHUNK, D), jnp.float32),
            pltpu.VMEM((CHUNK, D), jnp.float32),
            pltpu.VMEM_SHARED((n_nodes, D), jnp.float32),
            pltpu.SemaphoreType.DMA,
            pltpu.SemaphoreType.DMA,
            pltpu.SemaphoreType.DMA,
            pltpu.SemaphoreType.DMA,
            pltpu.SemaphoreType.DMA,
            pltpu.SemaphoreType.DMA,
            pltpu.SemaphoreType.DMA,
            pltpu.SemaphoreType.DMA,
            pltpu.SemaphoreType.DMA,
        ],
    )
    def gather_k(t2_hbm, t3_hbm, idx_hbm, g_hbm,
                 idx_v, a0, a1, a2, a3, t_sh,
                 sg0, sg1, sg2, sg3, sw0, sw1, sw2, sw3, si):
        cid = lax.axis_index("c")
        sid = lax.axis_index("s")

        roff = sid * rows_per_sub

        @pl.when(cid == 0)
        def _stage_t2():
            pltpu.sync_copy(t2_hbm.at[pl.ds(roff, rows_per_sub)],
                            t_sh.at[pl.ds(roff, rows_per_sub)])

        @pl.when(cid != 0)
        def _stage_t3():
            pltpu.sync_copy(t3_hbm.at[pl.ds(roff, rows_per_sub)],
                            t_sh.at[pl.ds(roff, rows_per_sub)])

        if tail_rows:
            toff = NS * rows_per_sub

            @pl.when((sid == 0) & (cid == 0))
            def _tail_t2():
                pltpu.sync_copy(t2_hbm.at[pl.ds(toff, tail_rows)],
                                t_sh.at[pl.ds(toff, tail_rows)])

            @pl.when((sid == 0) & (cid != 0))
            def _tail_t3():
                pltpu.sync_copy(t3_hbm.at[pl.ds(toff, tail_rows)],
                                t_sh.at[pl.ds(toff, tail_rows)])

        chunk0 = (cid * NS + sid) * chunks_per_sub
        n_blks = chunks_per_sub // IDXBLK
        pltpu.async_copy(idx_hbm.at[pl.ds(chunk0, IDXBLK)],
                         idx_v.at[pl.ds(0, IDXBLK)], si)
        plsc.subcore_barrier()

        slots = ((a0, sg0, sw0), (a1, sg1, sw1), (a2, sg2, sw2), (a3, sg3, sw3))
        steps_per_blk = IDXBLK // NBUF

        def blk_body(blk, carry):
            islot = (blk % 2) * IDXBLK
            pltpu.make_async_copy(idx_hbm.at[pl.ds(0, IDXBLK)],
                                  idx_v.at[pl.ds(0, IDXBLK)], si).wait()

            @pl.when(blk + 1 < n_blks)
            def _prefetch_next():
                nslot = ((blk + 1) % 2) * IDXBLK
                pltpu.async_copy(
                    idx_hbm.at[pl.ds(chunk0 + (blk + 1) * IDXBLK, IDXBLK)],
                    idx_v.at[pl.ds(nslot, IDXBLK)], si)

            def step(p, c2):
                for b, (buf, sg, sw) in enumerate(slots):
                    j = p * NBUF + b

                    @pl.when((blk > 0) | (p > 0))
                    def _wait_prev_write(buf=buf, sw=sw):
                        pltpu.make_async_copy(
                            g_hbm.at[pl.ds(0, CHUNK)], buf, sw).wait()

                    pltpu.async_copy(t_sh.at[idx_v.at[islot + j]], buf, sg)
                for b, (buf, sg, sw) in enumerate(slots):
                    j = p * NBUF + b
                    pltpu.make_async_copy(
                        g_hbm.at[pl.ds(0, CHUNK)], buf, sg).wait()
                    off = (chunk0 + blk * IDXBLK + j) * CHUNK
                    pltpu.async_copy(buf, g_hbm.at[pl.ds(off, CHUNK)], sw)
                return c2

            lax.fori_loop(0, steps_per_blk, step, 0)
            return carry

        lax.fori_loop(0, n_blks, blk_body, 0)
        for buf, _sg, sw in slots:
            pltpu.make_async_copy(g_hbm.at[pl.ds(0, CHUNK)], buf, sw).wait()

    return gather_k(t2, t3, idx2d)


def kernel(x, edge_attr, edge_index, W, b):
    n_nodes, d = x.shape
    n_edges = edge_attr.shape[0]
    senders = edge_index[0].astype(jnp.int32)
    receivers = edge_index[1].astype(jnp.int32)
    W1, W2, W3 = W[:d], W[d:2 * d], W[2 * d:]

    nb = 5
    node_rows = n_nodes // nb
    t2, t3 = pl.pallas_call(
        _node_tables_kernel,
        grid=(nb,),
        in_specs=[
            pl.BlockSpec((node_rows, d), lambda i: (i, 0)),
            pl.BlockSpec((d, d), lambda i: (0, 0)),
            pl.BlockSpec((d, d), lambda i: (0, 0)),
        ],
        out_specs=[
            pl.BlockSpec((node_rows, d), lambda i: (i, 0)),
            pl.BlockSpec((node_rows, d), lambda i: (i, 0)),
        ],
        out_shape=[jax.ShapeDtypeStruct((n_nodes, d), jnp.float32)] * 2,
    )(x, W2, W3)

    eb = 2560 // NPART
    ne_q = n_edges // NPART
    assert ne_q % eb == 0
    lcm = math.lcm(NS * CHUNK * IDXBLK, eb)
    half = -(-ne_q // lcm) * lcm
    pad = half - ne_q
    zpad = jnp.zeros((pad,), jnp.int32)
    nblk_q = ne_q // eb
    hblk = half // eb

    gs = []
    for q in range(NPART):
        s_q = lax.dynamic_slice_in_dim(senders, q * ne_q, ne_q)
        r_q = lax.dynamic_slice_in_dim(receivers, q * ne_q, ne_q)
        idx2d = jnp.concatenate([s_q, zpad, r_q, zpad]).reshape(-1, CHUNK)
        gs.append(_sc_gather(t2, t3, idx2d))

    out = None
    for q in range(NPART):
        g = gs[q]
        qb = q * nblk_q
        in_specs = [
            pl.BlockSpec((eb, d), lambda i, qb=qb: (i + qb, 0)),
            pl.BlockSpec((eb, d), lambda i: (i, 0)),
            pl.BlockSpec((eb, d), lambda i: (i + hblk, 0)),
            pl.BlockSpec((d, d), lambda i: (0, 0)),
            pl.BlockSpec((1, d), lambda i: (0, 0)),
        ]
        out_spec = pl.BlockSpec((eb, d), lambda i, qb=qb: (i + qb, 0))
        out_shape = jax.ShapeDtypeStruct((n_edges, d), jnp.float32)
        args = (edge_attr, g, g, W1, b.reshape(1, d))
        if q == 0:
            out = pl.pallas_call(
                _edge_out_kernel,
                grid=(nblk_q,),
                in_specs=in_specs,
                out_specs=out_spec,
                out_shape=out_shape,
            )(*args)
        else:
            out = pl.pallas_call(
                _edge_out_kernel_acc,
                grid=(nblk_q,),
                in_specs=[pl.BlockSpec(memory_space=pltpu.MemorySpace.HBM)]
                + in_specs,
                out_specs=out_spec,
                out_shape=out_shape,
                input_output_aliases={0: 0},
            )(out, *args)
    return out
